# baseline (speedup 1.0000x reference)
import functools
import jax, jax.numpy as jnp
from jax import lax
from jax.experimental import pallas as pl
from jax.experimental.pallas import tpu as pltpu
from jax.experimental.pallas import tpu_sc as plsc

B, OUT, N_CAT, VOCAB = 4096, 864, 26, 100000
BPW = 128

def _probe(r128, table128):
    mesh = plsc.VectorSubcoreMesh(core_axis_name="c", subcore_axis_name="s")
    @functools.partial(
        pl.kernel,
        out_type=jax.ShapeDtypeStruct((B * 27 // 4, 128), jnp.float32),
        scratch_types=[pltpu.VMEM((N_CAT, BPW), jnp.int32),
                       pltpu.VMEM((2, 64, 128), jnp.float32),
                       pltpu.SemaphoreType.DMA,
                       pltpu.SemaphoreType.DMA],
        mesh=mesh,
    )
    def enc(idx_hbm, tab_hbm, out_hbm, idx_v, rows_v, sem_a, sem_b):
        wid = lax.axis_index("s") * 2 + lax.axis_index("c")
        base = wid * BPW
        pltpu.sync_copy(idx_hbm.at[:, pl.ds(base, BPW)], idx_v)

        def fire(fld, buf, sem, jbase):
            return pltpu.async_copy(
                tab_hbm.at[idx_v.at[fld, pl.ds(jbase, 64)]],
                rows_v.at[buf], sem)

        def drain(buf, sem):
            pltpu.make_async_copy(tab_hbm.at[idx_v.at[0, pl.ds(0, 64)]],
                                  rows_v.at[buf], sem).wait()

        for h in (0, 1):
            jbase = h * 64
            fire(0, 0, sem_a, jbase)
            def pair(t, carry):
                f0 = 2 * t
                fire(f0 + 1, 1, sem_b, jbase)
                drain(0, sem_a)
                @pl.when(t < N_CAT // 2 - 1)
                def _():
                    fire(f0 + 2, 0, sem_a, jbase)
                drain(1, sem_b)
                return carry
            lax.fori_loop(0, N_CAT // 2, pair, 0)
        pltpu.sync_copy(rows_v.at[0], out_hbm.at[pl.ds(wid * 64, 64)])
    return enc(r128, table128)

def kernel(numeric, idx, table, W, b):
    idx = idx.astype(jnp.int32)
    r128 = (idx >> 2) + (jnp.arange(N_CAT, dtype=jnp.int32) * (VOCAB // 4))[:, None]
    table128 = table.reshape(650000, 128)
    return _probe(r128, table128).reshape(B, OUT)


# speedup vs baseline: 3.0943x; 3.0943x over previous
import functools

import jax
import jax.numpy as jnp
from jax import lax
from jax.experimental import pallas as pl
from jax.experimental.pallas import tpu as pltpu
from jax.experimental.pallas import tpu_sc as plsc

B = 4096
F_NUM = 13
N_CAT = 26
VOCAB = 100000
E = 32
P = 32
OUT = P + N_CAT * E

NC = 2
NS = 16
NW = NC * NS
BPW = B // NW

NG = 4
NUNITS = N_CAT * NG
BLK = 2048
NFULL = 48
LASTW = 1664
TAILBASE = NFULL * BLK + LASTW
TAILW = VOCAB - TAILBASE
NBOUND = 64

def _sc_encoder(rs, bs, st, tail, numeric_flat, tabT, W, b):
    mesh = plsc.VectorSubcoreMesh(core_axis_name="c", subcore_axis_name="s")

    @functools.partial(
        pl.kernel,
        out_type=(
            jax.ShapeDtypeStruct((N_CAT * E, B), jnp.float32),
            jax.ShapeDtypeStruct((B, P), jnp.float32),
        ),
        scratch_types=[
            pltpu.VMEM((B,), jnp.int32),
            pltpu.VMEM((B,), jnp.int32),
            pltpu.VMEM((NBOUND,), jnp.int32),
            pltpu.VMEM((8, TAILW), jnp.float32),
            pltpu.VMEM((8, BLK), jnp.float32),
            pltpu.VMEM((8, BLK), jnp.float32),
            pltpu.VMEM((8, B), jnp.float32),
            pltpu.VMEM((BPW * F_NUM + 16,), jnp.float32),
            pltpu.VMEM((F_NUM, P), jnp.float32),
            pltpu.VMEM((P,), jnp.float32),
            pltpu.VMEM((BPW, P), jnp.float32),
            pltpu.SemaphoreType.DMA,
            pltpu.SemaphoreType.DMA,
        ],
        mesh=mesh,
        compiler_params=pltpu.CompilerParams(needs_layout_passes=False),
    )
    def enc(rs_hbm, bs_hbm, st_hbm, tail_hbm, num_hbm, tab_hbm, w_hbm, b_hbm,
            embs_hbm, proj_hbm,
            rs_v, bs_v, st_v, tail_v, bufa_v, bufb_v, ob_v,
            num_v, w_v, b_v, proj_v, sem_a, sem_b):
        wid = lax.axis_index("s") * NC + lax.axis_index("c")
        base = wid * BPW
        IOTA16 = lax.iota(jnp.int32, 16)

        pltpu.sync_copy(num_hbm.at[pl.ds(base * F_NUM, BPW * F_NUM)],
                        num_v.at[pl.ds(0, BPW * F_NUM)])
        pltpu.sync_copy(w_hbm, w_v)
        pltpu.sync_copy(b_hbm, b_v)
        w_lo = [w_v[k, pl.ds(0, 16)] for k in range(F_NUM)]
        w_hi = [w_v[k, pl.ds(16, 16)] for k in range(F_NUM)]
        b_lo = b_v[pl.ds(0, 16)]
        b_hi = b_v[pl.ds(16, 16)]

        def prow(j, carry):
            v = num_v[pl.ds(j * F_NUM, 16)]
            a0 = b_lo
            a1 = b_hi
            for k in range(F_NUM):
                x = v[k]
                a0 = a0 + x * w_lo[k]
                a1 = a1 + x * w_hi[k]
            proj_v[j, pl.ds(0, 16)] = a0
            proj_v[j, pl.ds(16, 16)] = a1
            return carry
        lax.fori_loop(0, BPW, prow, 0)
        pltpu.sync_copy(proj_v, proj_hbm.at[pl.ds(base, BPW)])

        def do_unit(u):
            f = u // NG
            g = u - f * NG
            g8 = pl.multiple_of(g * 8, 8)

            pltpu.sync_copy(rs_hbm.at[pl.ds(f * B, B)], rs_v)
            pltpu.sync_copy(bs_hbm.at[pl.ds(f * B, B)], bs_v)
            pltpu.sync_copy(st_hbm.at[pl.ds(f * NBOUND, NBOUND)], st_v)
            pltpu.sync_copy(
                tail_hbm.at[pl.ds(pl.multiple_of((f * NG + g) * 8, 8), 8), :],
                tail_v)

            def fire(blk, buf_ref, sem):
                start = pl.multiple_of(blk * BLK, 128)
                return pltpu.async_copy(
                    tab_hbm.at[f, pl.ds(g8, 8), pl.ds(start, BLK)],
                    buf_ref, sem)

            def drain(buf_ref, sem):
                pltpu.make_async_copy(
                    tab_hbm.at[0, pl.ds(0, 8), pl.ds(0, BLK)],
                    buf_ref, sem).wait()

            def matches(blk, src_ref, src_w, rbase):
                pair = plsc.load_gather(
                    st_v, [jnp.minimum(blk + IOTA16, NBOUND - 1)])
                m0 = pair[0]
                m1 = pair[1]

                def grp(gi, carry):
                    mi = m0 + gi * 16 + IOTA16
                    msk = mi < m1
                    mic = jnp.minimum(mi, B - 1)
                    r16 = plsc.load_gather(rs_v, [mic]) - rbase
                    rl = jnp.clip(r16, 0, src_w - 1)
                    b16 = plsc.load_gather(bs_v, [mic])
                    for e in range(8):
                        ev = jnp.full((16,), e, dtype=jnp.int32)
                        v = plsc.load_gather(src_ref, [ev, rl])
                        plsc.store_scatter(ob_v, [ev, b16], v, mask=msk)
                    return carry
                lax.fori_loop(0, (m1 - m0 + 15) // 16, grp, 0)

            fire(0, bufa_v, sem_a)

            def pairloop(t, carry):
                blk0 = 2 * t
                fire(blk0 + 1, bufb_v, sem_b)
                drain(bufa_v, sem_a)
                matches(blk0, bufa_v, BLK, blk0 * BLK)

                @pl.when(t < NFULL // 2 - 1)
                def _():
                    fire(blk0 + 2, bufa_v, sem_a)

                drain(bufb_v, sem_b)
                matches(blk0 + 1, bufb_v, BLK, (blk0 + 1) * BLK)
                return carry
            lax.fori_loop(0, NFULL // 2, pairloop, 0)

            pltpu.async_copy(
                tab_hbm.at[f, pl.ds(g8, 8),
                           pl.ds(pl.multiple_of(NFULL * BLK, 128), LASTW)],
                bufa_v.at[pl.ds(0, 8), pl.ds(0, LASTW)], sem_a).wait()
            matches(jnp.int32(NFULL), bufa_v, LASTW, NFULL * BLK)

            matches(jnp.int32(NFULL + 1), tail_v, TAILW, TAILBASE)

            erow = pl.multiple_of(f * E + g * 8, 8)
            pltpu.sync_copy(ob_v, embs_hbm.at[pl.ds(erow, 8)])

        def uloop(j, carry):
            u = wid + j * NW

            @pl.when(u < NUNITS)
            def _():
                do_unit(u)
            return carry
        lax.fori_loop(0, 4, uloop, 0)

    return enc(rs, bs, st, tail, numeric_flat, tabT, W, b)


def kernel(numeric, idx, table, W, b):
    idx = idx.astype(jnp.int32)
    rs = jnp.sort(idx, axis=1)
    bs = jnp.argsort(idx, axis=1).astype(jnp.int32)
    bounds = jnp.concatenate([
        jnp.arange(0, (NFULL + 1) * BLK, BLK, dtype=jnp.int32),
        jnp.array([TAILBASE, VOCAB], dtype=jnp.int32),
        jnp.full((NBOUND - NFULL - 3,), VOCAB, dtype=jnp.int32),
    ])
    st = jax.vmap(
        lambda r: jnp.searchsorted(r, bounds).astype(jnp.int32))(rs)
    tail = table[:, TAILBASE:, :].transpose(0, 2, 1).reshape(
        N_CAT, NG, 8, TAILW)
    tabT = table.transpose(0, 2, 1)
    embsT, proj = _sc_encoder(
        rs.reshape(-1), bs.reshape(-1), st.reshape(-1),
        tail.reshape(N_CAT * NG * 8, TAILW),
        numeric.reshape(-1), tabT, W, b)
    return jnp.concatenate([proj, embsT.T], axis=1)


# speedup vs baseline: 3.7940x; 1.2261x over previous
import functools

import jax
import jax.numpy as jnp
from jax import lax
from jax.experimental import pallas as pl
from jax.experimental.pallas import tpu as pltpu
from jax.experimental.pallas import tpu_sc as plsc

B = 4096
F_NUM = 13
N_CAT = 26
VOCAB = 100000
E = 32
P = 32
OUT = P + N_CAT * E

NC = 2
NS = 16
NW = NC * NS
BPW = B // NW

NG = 4
NUNITS = N_CAT * NG
BLK = 2048
NFULL = 48
LASTW = 1664
TAILBASE = NFULL * BLK + LASTW
TAILW = VOCAB - TAILBASE
NBOUND = 64

def _sc_encoder(rs, bs, st, tail, numeric_flat, tabT, W, b):
    mesh = plsc.VectorSubcoreMesh(core_axis_name="c", subcore_axis_name="s")

    @functools.partial(
        pl.kernel,
        out_type=(
            jax.ShapeDtypeStruct((N_CAT * E, B), jnp.float32),
            jax.ShapeDtypeStruct((B, P), jnp.float32),
        ),
        scratch_types=[
            pltpu.VMEM((B,), jnp.int32),
            pltpu.VMEM((B,), jnp.int32),
            pltpu.VMEM((NBOUND,), jnp.int32),
            pltpu.VMEM((8, TAILW), jnp.float32),
            pltpu.VMEM((8, BLK), jnp.float32),
            pltpu.VMEM((8, BLK), jnp.float32),
            pltpu.VMEM((8, B), jnp.float32),
            pltpu.VMEM((BPW * F_NUM + 16,), jnp.float32),
            pltpu.VMEM((F_NUM, P), jnp.float32),
            pltpu.VMEM((P,), jnp.float32),
            pltpu.VMEM((BPW, P), jnp.float32),
            pltpu.SemaphoreType.DMA,
            pltpu.SemaphoreType.DMA,
        ],
        mesh=mesh,
        compiler_params=pltpu.CompilerParams(needs_layout_passes=False),
    )
    def enc(rs_hbm, bs_hbm, st_hbm, tail_hbm, num_hbm, tab_hbm, w_hbm, b_hbm,
            embs_hbm, proj_hbm,
            rs_v, bs_v, st_v, tail_v, bufa_v, bufb_v, ob_v,
            num_v, w_v, b_v, proj_v, sem_a, sem_b):
        wid = lax.axis_index("s") * NC + lax.axis_index("c")
        base = wid * BPW
        IOTA16 = lax.iota(jnp.int32, 16)

        pltpu.sync_copy(num_hbm.at[pl.ds(base * F_NUM, BPW * F_NUM)],
                        num_v.at[pl.ds(0, BPW * F_NUM)])
        pltpu.sync_copy(w_hbm, w_v)
        pltpu.sync_copy(b_hbm, b_v)
        w_lo = [w_v[k, pl.ds(0, 16)] for k in range(F_NUM)]
        w_hi = [w_v[k, pl.ds(16, 16)] for k in range(F_NUM)]
        b_lo = b_v[pl.ds(0, 16)]
        b_hi = b_v[pl.ds(16, 16)]

        def prow(j, carry):
            v = num_v[pl.ds(j * F_NUM, 16)]
            a0 = b_lo
            a1 = b_hi
            for k in range(F_NUM):
                x = v[k]
                a0 = a0 + x * w_lo[k]
                a1 = a1 + x * w_hi[k]
            proj_v[j, pl.ds(0, 16)] = a0
            proj_v[j, pl.ds(16, 16)] = a1
            return carry
        lax.fori_loop(0, BPW, prow, 0)
        pltpu.sync_copy(proj_v, proj_hbm.at[pl.ds(base, BPW)])

        def do_unit(u):
            f = u // NG
            g = u - f * NG
            g8 = pl.multiple_of(g * 8, 8)

            pltpu.sync_copy(rs_hbm.at[pl.ds(f * B, B)], rs_v)
            pltpu.sync_copy(bs_hbm.at[pl.ds(f * B, B)], bs_v)
            pltpu.sync_copy(st_hbm.at[pl.ds(f * NBOUND, NBOUND)], st_v)
            pltpu.sync_copy(
                tail_hbm.at[pl.ds(pl.multiple_of((f * NG + g) * 8, 8), 8), :],
                tail_v)

            def fire(blk, buf_ref, sem):
                start = pl.multiple_of(blk * BLK, 128)
                return pltpu.async_copy(
                    tab_hbm.at[f, pl.ds(g8, 8), pl.ds(start, BLK)],
                    buf_ref, sem)

            def drain(buf_ref, sem):
                pltpu.make_async_copy(
                    tab_hbm.at[0, pl.ds(0, 8), pl.ds(0, BLK)],
                    buf_ref, sem).wait()

            def matches(blk, src_ref, src_w, rbase):
                pair = plsc.load_gather(
                    st_v, [jnp.minimum(blk + IOTA16, NBOUND - 1)])
                m0 = pair[0]
                m1 = pair[1]

                def grp(gi, carry):
                    mi = m0 + gi * 16 + IOTA16
                    msk = mi < m1
                    mic = jnp.minimum(mi, B - 1)
                    r16 = plsc.load_gather(rs_v, [mic]) - rbase
                    rl = jnp.clip(r16, 0, src_w - 1)
                    b16 = plsc.load_gather(bs_v, [mic])
                    for e in range(8):
                        ev = jnp.full((16,), e, dtype=jnp.int32)
                        v = plsc.load_gather(src_ref, [ev, rl])
                        plsc.store_scatter(ob_v, [ev, b16], v, mask=msk)
                    return carry
                lax.fori_loop(0, (m1 - m0 + 15) // 16, grp, 0)

            fire(0, bufa_v, sem_a)

            def pairloop(t, carry):
                blk0 = 2 * t
                fire(blk0 + 1, bufb_v, sem_b)
                drain(bufa_v, sem_a)
                matches(blk0, bufa_v, BLK, blk0 * BLK)

                @pl.when(t < NFULL // 2 - 1)
                def _():
                    fire(blk0 + 2, bufa_v, sem_a)

                drain(bufb_v, sem_b)
                matches(blk0 + 1, bufb_v, BLK, (blk0 + 1) * BLK)
                return carry
            lax.fori_loop(0, NFULL // 2, pairloop, 0)

            pltpu.async_copy(
                tab_hbm.at[f, pl.ds(g8, 8),
                           pl.ds(pl.multiple_of(NFULL * BLK, 128), LASTW)],
                bufa_v.at[pl.ds(0, 8), pl.ds(0, LASTW)], sem_a).wait()
            matches(jnp.int32(NFULL), bufa_v, LASTW, NFULL * BLK)

            matches(jnp.int32(NFULL + 1), tail_v, TAILW, TAILBASE)

            erow = pl.multiple_of(f * E + g * 8, 8)
            pltpu.sync_copy(ob_v, embs_hbm.at[pl.ds(erow, 8)])

        def uloop(j, carry):
            u = wid + j * NW

            @pl.when(u < NUNITS)
            def _():
                do_unit(u)
            return carry
        lax.fori_loop(0, 4, uloop, 0)

    return enc(rs, bs, st, tail, numeric_flat, tabT, W, b)


def kernel(numeric, idx, table, W, b):
    idx = idx.astype(jnp.int32)
    rs = jnp.sort(idx, axis=1)
    bs = jnp.argsort(idx, axis=1).astype(jnp.int32)
    bid = jnp.where(idx < NFULL * BLK, idx >> 11,
                    jnp.where(idx < TAILBASE, NFULL, NFULL + 1))
    counts = jax.vmap(
        lambda r: jnp.bincount(r, length=NFULL + 2))(bid).astype(jnp.int32)
    st = jnp.concatenate([
        jnp.zeros((N_CAT, 1), jnp.int32),
        jnp.cumsum(counts, axis=1, dtype=jnp.int32),
        jnp.full((N_CAT, NBOUND - NFULL - 3), B, jnp.int32),
    ], axis=1)
    tail = table[:, TAILBASE:, :].transpose(0, 2, 1).reshape(
        N_CAT, NG, 8, TAILW)
    tabT = table.transpose(0, 2, 1)
    embsT, proj = _sc_encoder(
        rs.reshape(-1), bs.reshape(-1), st.reshape(-1),
        tail.reshape(N_CAT * NG * 8, TAILW),
        numeric.reshape(-1), tabT, W, b)
    return jnp.concatenate([proj, embsT.T], axis=1)


# speedup vs baseline: 4.0173x; 1.0589x over previous
import functools

import jax
import jax.numpy as jnp
from jax import lax
from jax.experimental import pallas as pl
from jax.experimental.pallas import tpu as pltpu
from jax.experimental.pallas import tpu_sc as plsc

B = 4096
F_NUM = 13
N_CAT = 26
VOCAB = 100000
E = 32
P = 32
OUT = P + N_CAT * E

NC = 2
NS = 16
NW = NC * NS
BPW = B // NW

NG = 4
NUNITS = N_CAT * NG
BLK = 4096
NFULL = 24
LASTW = 1664
TAILBASE = NFULL * BLK + LASTW
TAILW = VOCAB - TAILBASE
NBOUND = 64

def _sc_encoder(rs, bs, st, tail, numeric_flat, tabT, W, b):
    mesh = plsc.VectorSubcoreMesh(core_axis_name="c", subcore_axis_name="s")

    @functools.partial(
        pl.kernel,
        out_type=(
            jax.ShapeDtypeStruct((N_CAT * E, B), jnp.float32),
            jax.ShapeDtypeStruct((B, P), jnp.float32),
        ),
        scratch_types=[
            pltpu.VMEM((B,), jnp.int32),
            pltpu.VMEM((B,), jnp.int32),
            pltpu.VMEM((NBOUND,), jnp.int32),
            pltpu.VMEM((8, TAILW), jnp.float32),
            pltpu.VMEM((8, BLK), jnp.float32),
            pltpu.VMEM((8, BLK), jnp.float32),
            pltpu.VMEM((8, B), jnp.float32),
            pltpu.VMEM((BPW * F_NUM + 16,), jnp.float32),
            pltpu.VMEM((F_NUM, P), jnp.float32),
            pltpu.VMEM((P,), jnp.float32),
            pltpu.VMEM((BPW, P), jnp.float32),
            pltpu.SemaphoreType.DMA,
            pltpu.SemaphoreType.DMA,
        ],
        mesh=mesh,
        compiler_params=pltpu.CompilerParams(needs_layout_passes=False),
    )
    def enc(rs_hbm, bs_hbm, st_hbm, tail_hbm, num_hbm, tab_hbm, w_hbm, b_hbm,
            embs_hbm, proj_hbm,
            rs_v, bs_v, st_v, tail_v, bufa_v, bufb_v, ob_v,
            num_v, w_v, b_v, proj_v, sem_a, sem_b):
        wid = lax.axis_index("s") * NC + lax.axis_index("c")
        base = wid * BPW
        IOTA16 = lax.iota(jnp.int32, 16)

        pltpu.sync_copy(num_hbm.at[pl.ds(base * F_NUM, BPW * F_NUM)],
                        num_v.at[pl.ds(0, BPW * F_NUM)])
        pltpu.sync_copy(w_hbm, w_v)
        pltpu.sync_copy(b_hbm, b_v)
        w_lo = [w_v[k, pl.ds(0, 16)] for k in range(F_NUM)]
        w_hi = [w_v[k, pl.ds(16, 16)] for k in range(F_NUM)]
        b_lo = b_v[pl.ds(0, 16)]
        b_hi = b_v[pl.ds(16, 16)]

        def prow(j, carry):
            v = num_v[pl.ds(j * F_NUM, 16)]
            a0 = b_lo
            a1 = b_hi
            for k in range(F_NUM):
                x = v[k]
                a0 = a0 + x * w_lo[k]
                a1 = a1 + x * w_hi[k]
            proj_v[j, pl.ds(0, 16)] = a0
            proj_v[j, pl.ds(16, 16)] = a1
            return carry
        lax.fori_loop(0, BPW, prow, 0)
        pltpu.sync_copy(proj_v, proj_hbm.at[pl.ds(base, BPW)])

        def do_unit(u):
            f = u // NG
            g = u - f * NG
            g8 = pl.multiple_of(g * 8, 8)

            pltpu.sync_copy(rs_hbm.at[pl.ds(f * B, B)], rs_v)
            pltpu.sync_copy(bs_hbm.at[pl.ds(f * B, B)], bs_v)
            pltpu.sync_copy(st_hbm.at[pl.ds(f * NBOUND, NBOUND)], st_v)
            pltpu.sync_copy(
                tail_hbm.at[pl.ds(pl.multiple_of((f * NG + g) * 8, 8), 8), :],
                tail_v)

            def fire(blk, buf_ref, sem):
                start = pl.multiple_of(blk * BLK, 128)
                return pltpu.async_copy(
                    tab_hbm.at[f, pl.ds(g8, 8), pl.ds(start, BLK)],
                    buf_ref, sem)

            def drain(buf_ref, sem):
                pltpu.make_async_copy(
                    tab_hbm.at[0, pl.ds(0, 8), pl.ds(0, BLK)],
                    buf_ref, sem).wait()

            def matches(blk, src_ref, src_w, rbase):
                pair = plsc.load_gather(
                    st_v, [jnp.minimum(blk + IOTA16, NBOUND - 1)])
                m0 = pair[0]
                m1 = pair[1]

                def grp(gi, carry):
                    mi = m0 + gi * 16 + IOTA16
                    msk = mi < m1
                    mic = jnp.minimum(mi, B - 1)
                    r16 = plsc.load_gather(rs_v, [mic]) - rbase
                    rl = jnp.clip(r16, 0, src_w - 1)
                    b16 = plsc.load_gather(bs_v, [mic])
                    for e in range(8):
                        ev = jnp.full((16,), e, dtype=jnp.int32)
                        v = plsc.load_gather(src_ref, [ev, rl])
                        plsc.store_scatter(ob_v, [ev, b16], v, mask=msk)
                    return carry
                lax.fori_loop(0, (m1 - m0 + 15) // 16, grp, 0)

            fire(0, bufa_v, sem_a)

            def pairloop(t, carry):
                blk0 = 2 * t
                fire(blk0 + 1, bufb_v, sem_b)
                drain(bufa_v, sem_a)
                matches(blk0, bufa_v, BLK, blk0 * BLK)

                @pl.when(t < NFULL // 2 - 1)
                def _():
                    fire(blk0 + 2, bufa_v, sem_a)

                drain(bufb_v, sem_b)
                matches(blk0 + 1, bufb_v, BLK, (blk0 + 1) * BLK)
                return carry
            lax.fori_loop(0, NFULL // 2, pairloop, 0)

            pltpu.async_copy(
                tab_hbm.at[f, pl.ds(g8, 8),
                           pl.ds(pl.multiple_of(NFULL * BLK, 128), LASTW)],
                bufa_v.at[pl.ds(0, 8), pl.ds(0, LASTW)], sem_a).wait()
            matches(jnp.int32(NFULL), bufa_v, LASTW, NFULL * BLK)

            matches(jnp.int32(NFULL + 1), tail_v, TAILW, TAILBASE)

            erow = pl.multiple_of(f * E + g * 8, 8)
            pltpu.sync_copy(ob_v, embs_hbm.at[pl.ds(erow, 8)])

        def uloop(j, carry):
            u = wid + j * NW

            @pl.when(u < NUNITS)
            def _():
                do_unit(u)
            return carry
        lax.fori_loop(0, 4, uloop, 0)

    return enc(rs, bs, st, tail, numeric_flat, tabT, W, b)


def kernel(numeric, idx, table, W, b):
    idx = idx.astype(jnp.int32)
    rs = jnp.sort(idx, axis=1)
    bs = jnp.argsort(idx, axis=1).astype(jnp.int32)
    bid = jnp.where(idx < NFULL * BLK, idx >> 12,
                    jnp.where(idx < TAILBASE, NFULL, NFULL + 1))
    counts = jax.vmap(
        lambda r: jnp.bincount(r, length=NFULL + 2))(bid).astype(jnp.int32)
    st = jnp.concatenate([
        jnp.zeros((N_CAT, 1), jnp.int32),
        jnp.cumsum(counts, axis=1, dtype=jnp.int32),
        jnp.full((N_CAT, NBOUND - NFULL - 3), B, jnp.int32),
    ], axis=1)
    tail = table[:, TAILBASE:, :].transpose(0, 2, 1).reshape(
        N_CAT, NG, 8, TAILW)
    tabT = table.transpose(0, 2, 1)
    embsT, proj = _sc_encoder(
        rs.reshape(-1), bs.reshape(-1), st.reshape(-1),
        tail.reshape(N_CAT * NG * 8, TAILW),
        numeric.reshape(-1), tabT, W, b)
    return jnp.concatenate([proj, embsT.T], axis=1)


# speedup vs baseline: 4.1269x; 1.0273x over previous
import functools

import jax
import jax.numpy as jnp
from jax import lax
from jax.experimental import pallas as pl
from jax.experimental.pallas import tpu as pltpu
from jax.experimental.pallas import tpu_sc as plsc

B = 4096
F_NUM = 13
N_CAT = 26
VOCAB = 100000
E = 32
P = 32
OUT = P + N_CAT * E

NC = 2
NS = 16
NW = NC * NS
BPW = B // NW

NG = 4
NUNITS = N_CAT * NG
BLK = 4096
NFULL = 24
LASTW = 1664
TAILBASE = NFULL * BLK + LASTW
TAILW = VOCAB - TAILBASE
NBOUND = 64

def _sc_encoder(rs, bs, st, tail, numeric_flat, tabT, W, b):
    mesh = plsc.VectorSubcoreMesh(core_axis_name="c", subcore_axis_name="s")

    @functools.partial(
        pl.kernel,
        out_type=(
            jax.ShapeDtypeStruct((N_CAT * E, B), jnp.float32),
            jax.ShapeDtypeStruct((B, P), jnp.float32),
        ),
        scratch_types=[
            pltpu.VMEM((B,), jnp.int32),
            pltpu.VMEM((B,), jnp.int32),
            pltpu.VMEM((NBOUND,), jnp.int32),
            pltpu.VMEM((8, TAILW), jnp.float32),
            pltpu.VMEM((8, BLK), jnp.float32),
            pltpu.VMEM((8, BLK), jnp.float32),
            pltpu.VMEM((8, B), jnp.float32),
            pltpu.VMEM((BPW * F_NUM + 16,), jnp.float32),
            pltpu.VMEM((F_NUM, P), jnp.float32),
            pltpu.VMEM((P,), jnp.float32),
            pltpu.VMEM((BPW, P), jnp.float32),
            pltpu.SemaphoreType.DMA,
            pltpu.SemaphoreType.DMA,
        ],
        mesh=mesh,
        compiler_params=pltpu.CompilerParams(needs_layout_passes=False),
    )
    def enc(rs_hbm, bs_hbm, st_hbm, tail_hbm, num_hbm, tab_hbm, w_hbm, b_hbm,
            embs_hbm, proj_hbm,
            rs_v, bs_v, st_v, tail_v, bufa_v, bufb_v, ob_v,
            num_v, w_v, b_v, proj_v, sem_a, sem_b):
        wid = lax.axis_index("s") * NC + lax.axis_index("c")
        base = wid * BPW
        IOTA16 = lax.iota(jnp.int32, 16)

        pltpu.sync_copy(num_hbm.at[pl.ds(base * F_NUM, BPW * F_NUM)],
                        num_v.at[pl.ds(0, BPW * F_NUM)])
        pltpu.sync_copy(w_hbm, w_v)
        pltpu.sync_copy(b_hbm, b_v)
        w_lo = [w_v[k, pl.ds(0, 16)] for k in range(F_NUM)]
        w_hi = [w_v[k, pl.ds(16, 16)] for k in range(F_NUM)]
        b_lo = b_v[pl.ds(0, 16)]
        b_hi = b_v[pl.ds(16, 16)]

        def prow(j, carry):
            v = num_v[pl.ds(j * F_NUM, 16)]
            a0 = b_lo
            a1 = b_hi
            for k in range(F_NUM):
                x = v[k]
                a0 = a0 + x * w_lo[k]
                a1 = a1 + x * w_hi[k]
            proj_v[j, pl.ds(0, 16)] = a0
            proj_v[j, pl.ds(16, 16)] = a1
            return carry
        lax.fori_loop(0, BPW, prow, 0)
        pltpu.sync_copy(proj_v, proj_hbm.at[pl.ds(base, BPW)])

        def do_unit(u):
            f = u // NG
            g = u - f * NG
            g8 = pl.multiple_of(g * 8, 8)

            pltpu.sync_copy(rs_hbm.at[pl.ds(f * B, B)], rs_v)
            pltpu.sync_copy(bs_hbm.at[pl.ds(f * B, B)], bs_v)
            pltpu.sync_copy(st_hbm.at[pl.ds(f * NBOUND, NBOUND)], st_v)
            pltpu.sync_copy(
                tail_hbm.at[pl.ds(pl.multiple_of((f * NG + g) * 8, 8), 8), :],
                tail_v)

            def fire(blk, buf_ref, sem):
                start = pl.multiple_of(blk * BLK, 128)
                return pltpu.async_copy(
                    tab_hbm.at[f, pl.ds(g8, 8), pl.ds(start, BLK)],
                    buf_ref, sem)

            def drain(buf_ref, sem):
                pltpu.make_async_copy(
                    tab_hbm.at[0, pl.ds(0, 8), pl.ds(0, BLK)],
                    buf_ref, sem).wait()

            def matches(blk, src_ref, src_w, rbase):
                pair = plsc.load_gather(
                    st_v, [jnp.minimum(blk + IOTA16, NBOUND - 1)])
                m0 = pair[0]
                m1 = pair[1]

                def grp(gi, carry):
                    mi = m0 + gi * 16 + IOTA16
                    msk = mi < m1
                    mic = jnp.minimum(mi, B - 1)
                    r16 = plsc.load_gather(rs_v, [mic]) - rbase
                    rl = jnp.clip(r16, 0, src_w - 1)
                    b16 = plsc.load_gather(bs_v, [mic])
                    for e in range(8):
                        ev = jnp.full((16,), e, dtype=jnp.int32)
                        v = plsc.load_gather(src_ref, [ev, rl])
                        plsc.store_scatter(ob_v, [ev, b16], v, mask=msk)
                    return carry
                lax.fori_loop(0, (m1 - m0 + 15) // 16, grp, 0)

            fire(0, bufa_v, sem_a)

            def pairloop(t, carry):
                blk0 = 2 * t
                fire(blk0 + 1, bufb_v, sem_b)
                drain(bufa_v, sem_a)
                matches(blk0, bufa_v, BLK, blk0 * BLK)

                @pl.when(t < NFULL // 2 - 1)
                def _():
                    fire(blk0 + 2, bufa_v, sem_a)

                drain(bufb_v, sem_b)
                matches(blk0 + 1, bufb_v, BLK, (blk0 + 1) * BLK)
                return carry
            lax.fori_loop(0, NFULL // 2, pairloop, 0)

            pltpu.async_copy(
                tab_hbm.at[f, pl.ds(g8, 8),
                           pl.ds(pl.multiple_of(NFULL * BLK, 128), LASTW)],
                bufa_v.at[pl.ds(0, 8), pl.ds(0, LASTW)], sem_a).wait()
            matches(jnp.int32(NFULL), bufa_v, LASTW, NFULL * BLK)

            matches(jnp.int32(NFULL + 1), tail_v, TAILW, TAILBASE)

            erow = pl.multiple_of(f * E + g * 8, 8)
            pltpu.sync_copy(ob_v, embs_hbm.at[pl.ds(erow, 8)])

        def uloop(j, carry):
            u = wid + j * NW

            @pl.when(u < NUNITS)
            def _():
                do_unit(u)
            return carry
        lax.fori_loop(0, 4, uloop, 0)

    return enc(rs, bs, st, tail, numeric_flat, tabT, W, b)


def kernel(numeric, idx, table, W, b):
    idx = idx.astype(jnp.int32)
    rs = jnp.sort(idx, axis=1)
    bs = jnp.argsort(idx, axis=1).astype(jnp.int32)
    bounds = jnp.concatenate([
        jnp.arange(0, (NFULL + 1) * BLK, BLK, dtype=jnp.int32),
        jnp.array([TAILBASE, VOCAB], dtype=jnp.int32),
        jnp.full((NBOUND - NFULL - 3,), VOCAB, dtype=jnp.int32),
    ])
    st = jnp.sum(idx[:, :, None] < bounds[None, None, :],
                 axis=1, dtype=jnp.int32)
    tail = table[:, TAILBASE:, :].transpose(0, 2, 1).reshape(
        N_CAT, NG, 8, TAILW)
    tabT = table.transpose(0, 2, 1)
    embsT, proj = _sc_encoder(
        rs.reshape(-1), bs.reshape(-1), st.reshape(-1),
        tail.reshape(N_CAT * NG * 8, TAILW),
        numeric.reshape(-1), tabT, W, b)
    return jnp.concatenate([proj, embsT.T], axis=1)
